# Initial kernel scaffold; baseline (speedup 1.0000x reference)
import jax
import jax.numpy as jnp
from jax.experimental import pallas as pl


def kernel(x, edge_index, W1, b1, W2, b2):
    raise NotImplementedError("write your pallas kernel here")


# speedup vs baseline: 34.6171x; 34.6171x over previous
import functools

import jax
import jax.numpy as jnp
from jax import lax
from jax.experimental import pallas as pl
from jax.experimental.pallas import tpu as pltpu
from jax.experimental.pallas import tpu_sc as plsc

N = 10000
NP = 10240
E = 320000
D_IN, D_HID, D_OUT = 128, 64, 2
NC, NS = 2, 16
NW = NC * NS
EPW = E // NW
CH = 125
NCH = EPW // CH
RPT = NP // NS

_MESH = plsc.VectorSubcoreMesh(
    core_axis_name="c", subcore_axis_name="s", num_cores=NC, num_subcores=NS)
_SC_PARAMS = pltpu.CompilerParams(use_tc_tiling_on_sc=False)


def _deg_body(dst_hbm, ones_hbm, zeros_hbm, deg_out, idx_v, ones_v, acc):
  c = lax.axis_index("c")
  s = lax.axis_index("s")
  w = c * NS + s
  pltpu.sync_copy(dst_hbm.at[w], idx_v)
  pltpu.sync_copy(ones_hbm, ones_v)
  pltpu.sync_copy(zeros_hbm.at[pl.ds(s * RPT, RPT)],
                  acc.at[pl.ds(s * RPT, RPT)])
  plsc.subcore_barrier()

  def body(j, carry):
    pltpu.sync_copy(ones_v, acc.at[idx_v.at[j]], add=True)
    return carry

  lax.fori_loop(0, NCH, body, 0)
  plsc.subcore_barrier()
  pltpu.sync_copy(acc.at[pl.ds(s * RPT, RPT)],
                  deg_out.at[c].at[pl.ds(s * RPT, RPT)])


_deg_call = functools.partial(
    pl.kernel,
    out_type=jax.ShapeDtypeStruct((NC, NP), jnp.float32),
    mesh=_MESH,
    compiler_params=_SC_PARAMS,
    scratch_types=[
        pltpu.VMEM((NCH, CH), jnp.int32),
        pltpu.VMEM((CH,), jnp.float32),
        pltpu.VMEM_SHARED((NP,), jnp.float32),
    ],
)(_deg_body)


def _make_agg(d):
  def _agg_body(hs_hbm, src_hbm, dst_hbm, out_hbm,
                idxs_v, idxd_v, rows0, rows1, gsem0, gsem1, acc):
    c = lax.axis_index("c")
    s = lax.axis_index("s")
    w = c * NS + s
    pltpu.sync_copy(src_hbm.at[w], idxs_v)
    pltpu.sync_copy(dst_hbm.at[w], idxd_v)
    pltpu.sync_copy(hs_hbm.at[pl.ds(s * RPT, RPT)],
                    acc.at[pl.ds(s * RPT, RPT)])
    plsc.subcore_barrier()

    pltpu.async_copy(hs_hbm.at[idxs_v.at[0]], rows0, gsem0)

    def body(t, carry):
      j = 2 * t
      pltpu.make_async_copy(hs_hbm.at[idxs_v.at[j]], rows0, gsem0).wait()
      pltpu.async_copy(hs_hbm.at[idxs_v.at[j + 1]], rows1, gsem1)
      pltpu.sync_copy(rows0, acc.at[idxd_v.at[j]], add=True)
      pltpu.make_async_copy(hs_hbm.at[idxs_v.at[j + 1]], rows1, gsem1).wait()

      @pl.when(j + 2 < NCH)
      def _():
        pltpu.async_copy(hs_hbm.at[idxs_v.at[j + 2]], rows0, gsem0)

      pltpu.sync_copy(rows1, acc.at[idxd_v.at[j + 1]], add=True)
      return carry

    lax.fori_loop(0, NCH // 2, body, 0)
    plsc.subcore_barrier()
    pltpu.sync_copy(acc.at[pl.ds(s * RPT, RPT)],
                    out_hbm.at[c].at[pl.ds(s * RPT, RPT)])

  return functools.partial(
      pl.kernel,
      out_type=jax.ShapeDtypeStruct((NC, NP, d), jnp.float32),
      mesh=_MESH,
      compiler_params=_SC_PARAMS,
      scratch_types=[
          pltpu.VMEM((NCH, CH), jnp.int32),
          pltpu.VMEM((NCH, CH), jnp.int32),
          pltpu.VMEM((CH, d), jnp.float32),
          pltpu.VMEM((CH, d), jnp.float32),
          pltpu.SemaphoreType.DMA,
          pltpu.SemaphoreType.DMA,
          pltpu.VMEM_SHARED((NP, d), jnp.float32),
      ],
  )(_agg_body)


D_AGG2 = 16
_agg64_call = _make_agg(D_HID)
_agg2_call = _make_agg(D_AGG2)


def _mm1_body(x_ref, w1_ref, degp_ref, o_ref):
  dis = lax.rsqrt(degp_ref[0] + degp_ref[1] + 1.0)
  o_ref[...] = jnp.dot(x_ref[...], w1_ref[...],
                       preferred_element_type=jnp.float32) * dis


_mm1_call = pl.pallas_call(
    _mm1_body,
    out_shape=jax.ShapeDtypeStruct((NP, D_HID), jnp.float32),
)


def _mm2_body(degp_ref, hs1_ref, cp_ref, b1_ref, w2_ref, o_ref):
  dis = lax.rsqrt(degp_ref[0] + degp_ref[1] + 1.0)
  p = cp_ref[0] + cp_ref[1] - hs1_ref[...]
  h1 = jnp.maximum(dis * p + b1_ref[...], 0.0)
  o_ref[...] = jnp.dot(h1, w2_ref[...],
                       preferred_element_type=jnp.float32) * dis


_mm2_call = pl.pallas_call(
    _mm2_body,
    out_shape=jax.ShapeDtypeStruct((NP, D_AGG2), jnp.float32),
)


def _soft_body(degp_ref, hs2_ref, qp_ref, b2_ref, o_ref):
  dis = lax.rsqrt(degp_ref[0] + degp_ref[1] + 1.0)
  z = dis * (qp_ref[0] + qp_ref[1] - hs2_ref[...]) + b2_ref[...]
  z = z[:, :D_OUT]
  m = jnp.max(z, axis=-1, keepdims=True)
  e = jnp.exp(z - m)
  o_ref[...] = e / jnp.sum(e, axis=-1, keepdims=True)


_soft_call = pl.pallas_call(
    _soft_body,
    out_shape=jax.ShapeDtypeStruct((NP, D_OUT), jnp.float32),
)


def kernel(x, edge_index, W1, b1, W2, b2):
  ei = edge_index.astype(jnp.int32)
  src3 = ei[0].reshape(NW, NCH, CH)
  dst3 = ei[1].reshape(NW, NCH, CH)
  xp = jnp.zeros((NP, D_IN), jnp.float32).at[:N].set(x)
  ones_c = jnp.ones((CH,), jnp.float32)
  zeros_np = jnp.zeros((NP,), jnp.float32)

  degp = _deg_call(dst3, ones_c, zeros_np)
  degp3 = degp[..., None]
  hs1 = _mm1_call(xp, W1, degp3)
  cp = _agg64_call(hs1, src3, dst3)
  w2p = jnp.zeros((D_HID, D_AGG2), jnp.float32).at[:, :D_OUT].set(W2)
  b2p = jnp.zeros((1, D_AGG2), jnp.float32).at[:, :D_OUT].set(b2)
  hs2 = _mm2_call(degp3, hs1, cp, b1.reshape(1, -1), w2p)
  qp = _agg2_call(hs2, src3, dst3)
  out = _soft_call(degp3, hs2, qp, b2p)
  return out[:N]


# speedup vs baseline: 46.8008x; 1.3520x over previous
import functools

import jax
import jax.numpy as jnp
from jax import lax
from jax.experimental import pallas as pl
from jax.experimental.pallas import tpu as pltpu
from jax.experimental.pallas import tpu_sc as plsc

N = 10000
NP = 10240
E = 320000
D_IN, D_HID, D_OUT = 128, 64, 2
NC, NS = 2, 16
NW = NC * NS
EPW = E // NW
CH = 125
NCH = EPW // CH
RPT = NP // NS

_MESH = plsc.VectorSubcoreMesh(
    core_axis_name="c", subcore_axis_name="s", num_cores=NC, num_subcores=NS)
_SC_PARAMS = pltpu.CompilerParams(use_tc_tiling_on_sc=False)


def _deg_body(dst_hbm, ones_hbm, zeros_hbm, deg_out, idx_v, ones_v, acc):
  c = lax.axis_index("c")
  s = lax.axis_index("s")
  w = c * NS + s
  pltpu.sync_copy(dst_hbm.at[w], idx_v)
  pltpu.sync_copy(ones_hbm, ones_v)
  pltpu.sync_copy(zeros_hbm.at[pl.ds(s * RPT, RPT)],
                  acc.at[pl.ds(s * RPT, RPT)])
  plsc.subcore_barrier()

  def body(j, carry):
    pltpu.sync_copy(ones_v, acc.at[idx_v.at[j]], add=True)
    return carry

  lax.fori_loop(0, NCH, body, 0)
  plsc.subcore_barrier()
  pltpu.sync_copy(acc.at[pl.ds(s * RPT, RPT)],
                  deg_out.at[c].at[pl.ds(s * RPT, RPT)])


_deg_call = functools.partial(
    pl.kernel,
    out_type=jax.ShapeDtypeStruct((NC, NP), jnp.float32),
    mesh=_MESH,
    compiler_params=_SC_PARAMS,
    scratch_types=[
        pltpu.VMEM((NCH, CH), jnp.int32),
        pltpu.VMEM((CH,), jnp.float32),
        pltpu.VMEM_SHARED((NP,), jnp.float32),
    ],
)(_deg_body)


NBUF = 8


def _make_agg(d):
  def _agg_body(hs_hbm, src_hbm, dst_hbm, out_hbm,
                idxs_v, idxd_v, rows, gsems, ssems, acc):
    c = lax.axis_index("c")
    s = lax.axis_index("s")
    w = c * NS + s
    pltpu.sync_copy(src_hbm.at[w], idxs_v)
    pltpu.sync_copy(dst_hbm.at[w], idxd_v)
    pltpu.sync_copy(hs_hbm.at[pl.ds(s * RPT, RPT)],
                    acc.at[pl.ds(s * RPT, RPT)])
    plsc.subcore_barrier()

    half = NBUF // 2
    for b in range(half):
      pltpu.async_copy(hs_hbm.at[idxs_v.at[b]], rows[b], gsems[b])

    def body(t, carry):
      for b in range(NBUF):
        j = NBUF * t + b
        bg = (b + half) % NBUF
        pltpu.make_async_copy(hs_hbm.at[idxs_v.at[j]], rows[b],
                              gsems[b]).wait()
        pltpu.async_copy(rows[b], acc.at[idxd_v.at[j]], ssems[b], add=True)
        @pl.when(j >= half)
        def _():
          pltpu.make_async_copy(rows[bg], acc.at[idxd_v.at[j - half]],
                                ssems[bg]).wait()

        @pl.when(j + half < NCH)
        def _():
          pltpu.async_copy(hs_hbm.at[idxs_v.at[j + half]], rows[bg],
                           gsems[bg])
      return carry

    lax.fori_loop(0, NCH // NBUF, body, 0)
    for b in range(half, NBUF):
      j = NCH - NBUF + b
      pltpu.make_async_copy(rows[b], acc.at[idxd_v.at[j]], ssems[b]).wait()
    plsc.subcore_barrier()
    pltpu.sync_copy(acc.at[pl.ds(s * RPT, RPT)],
                    out_hbm.at[c].at[pl.ds(s * RPT, RPT)])

  return functools.partial(
      pl.kernel,
      out_type=jax.ShapeDtypeStruct((NC, NP, d), jnp.float32),
      mesh=_MESH,
      compiler_params=_SC_PARAMS,
      scratch_types=[
          pltpu.VMEM((NCH, CH), jnp.int32),
          pltpu.VMEM((NCH, CH), jnp.int32),
          [pltpu.VMEM((CH, d), jnp.float32)] * NBUF,
          [pltpu.SemaphoreType.DMA] * NBUF,
          [pltpu.SemaphoreType.DMA] * NBUF,
          pltpu.VMEM_SHARED((NP, d), jnp.float32),
      ],
  )(_agg_body)


D_AGG2 = 16
_agg64_call = _make_agg(D_HID)
_agg2_call = _make_agg(D_AGG2)


def _mm1_body(x_ref, w1_ref, degp_ref, o_ref):
  dis = lax.rsqrt(degp_ref[0] + degp_ref[1] + 1.0)
  o_ref[...] = jnp.dot(x_ref[...], w1_ref[...],
                       preferred_element_type=jnp.float32) * dis


_mm1_call = pl.pallas_call(
    _mm1_body,
    out_shape=jax.ShapeDtypeStruct((NP, D_HID), jnp.float32),
)


def _mm2_body(degp_ref, hs1_ref, cp_ref, b1_ref, w2_ref, o_ref):
  dis = lax.rsqrt(degp_ref[0] + degp_ref[1] + 1.0)
  p = cp_ref[0] + cp_ref[1] - hs1_ref[...]
  h1 = jnp.maximum(dis * p + b1_ref[...], 0.0)
  o_ref[...] = jnp.dot(h1, w2_ref[...],
                       preferred_element_type=jnp.float32) * dis


_mm2_call = pl.pallas_call(
    _mm2_body,
    out_shape=jax.ShapeDtypeStruct((NP, D_AGG2), jnp.float32),
)


def _soft_body(degp_ref, hs2_ref, qp_ref, b2_ref, o_ref):
  dis = lax.rsqrt(degp_ref[0] + degp_ref[1] + 1.0)
  z = dis * (qp_ref[0] + qp_ref[1] - hs2_ref[...]) + b2_ref[...]
  z = z[:, :D_OUT]
  m = jnp.max(z, axis=-1, keepdims=True)
  e = jnp.exp(z - m)
  o_ref[...] = e / jnp.sum(e, axis=-1, keepdims=True)


_soft_call = pl.pallas_call(
    _soft_body,
    out_shape=jax.ShapeDtypeStruct((NP, D_OUT), jnp.float32),
)


def kernel(x, edge_index, W1, b1, W2, b2):
  ei = edge_index.astype(jnp.int32)
  src3 = ei[0].reshape(NW, NCH, CH)
  dst3 = ei[1].reshape(NW, NCH, CH)
  xp = jnp.zeros((NP, D_IN), jnp.float32).at[:N].set(x)
  ones_c = jnp.ones((CH,), jnp.float32)
  zeros_np = jnp.zeros((NP,), jnp.float32)

  degp = _deg_call(dst3, ones_c, zeros_np)
  degp3 = degp[..., None]
  hs1 = _mm1_call(xp, W1, degp3)
  cp = _agg64_call(hs1, src3, dst3)
  w2p = jnp.zeros((D_HID, D_AGG2), jnp.float32).at[:, :D_OUT].set(W2)
  b2p = jnp.zeros((1, D_AGG2), jnp.float32).at[:, :D_OUT].set(b2)
  hs2 = _mm2_call(degp3, hs1, cp, b1.reshape(1, -1), w2p)
  qp = _agg2_call(hs2, src3, dst3)
  out = _soft_call(degp3, hs2, qp, b2p)
  return out[:N]
